# baseline (speedup 1.0000x reference)
import functools

import jax
import jax.numpy as jnp
from jax import lax
from jax.experimental import pallas as pl
from jax.experimental.pallas import tpu as pltpu
from jax.experimental.pallas import tpu_sc as plsc

BATCH = 16384
DIM = 32
NC = 2
NS = 16
LANES = 16
NW = NC * NS
BPW = BATCH // NW
IDX_CHUNK = 128
NCHUNK = BPW // IDX_CHUNK

_mesh = plsc.VectorSubcoreMesh(core_axis_name="c", subcore_axis_name="s")


@functools.partial(
    pl.kernel,
    out_type=jax.ShapeDtypeStruct((BATCH,), jnp.float32),
    mesh=_mesh,
    compiler_params=pltpu.CompilerParams(
        needs_layout_passes=False, use_tc_tiling_on_sc=False),
    scratch_types=[
        pltpu.VMEM((NCHUNK, IDX_CHUNK), jnp.int32),
        pltpu.VMEM((NCHUNK, IDX_CHUNK), jnp.int32),
        pltpu.VMEM((BPW, DIM), jnp.float32),
        pltpu.VMEM((BPW, DIM), jnp.float32),
        pltpu.VMEM((BPW,), jnp.float32),
        pltpu.SemaphoreType.DMA,
        pltpu.SemaphoreType.DMA,
        pltpu.SemaphoreType.DMA,
    ],
)
def _mf_sc(user_hbm, item_hbm, utab_hbm, itab_hbm, out_hbm,
           uidx_v, iidx_v, urows_v, irows_v, res_v,
           sem_idx, sem_u, sem_i):
    wid = lax.axis_index("s") * NC + lax.axis_index("c")
    base = wid * BPW

    idx_copies = []
    for j in range(NCHUNK):
        off = base + j * IDX_CHUNK
        idx_copies.append(pltpu.async_copy(
            user_hbm.at[pl.ds(off, IDX_CHUNK)], uidx_v.at[j], sem_idx))
        idx_copies.append(pltpu.async_copy(
            item_hbm.at[pl.ds(off, IDX_CHUNK)], iidx_v.at[j], sem_idx))
    for c in idx_copies:
        c.wait()

    row_copies = []
    for j in range(NCHUNK):
        dst = pl.ds(j * IDX_CHUNK, IDX_CHUNK)
        row_copies.append(pltpu.async_copy(
            utab_hbm.at[uidx_v.at[j]], urows_v.at[dst], sem_u))
        row_copies.append(pltpu.async_copy(
            itab_hbm.at[iidx_v.at[j]], irows_v.at[dst], sem_i))
    for c in row_copies:
        c.wait()

    lane = lax.iota(jnp.int32, LANES)

    def chunk_body(cix, carry):
        b = cix * LANES + lane
        acc = jnp.zeros((LANES,), jnp.float32)
        for d in range(DIM):
            dcol = jnp.full((LANES,), d, jnp.int32)
            u = plsc.load_gather(urows_v, [b, dcol])
            v = plsc.load_gather(irows_v, [b, dcol])
            acc = acc + u * v
        plsc.store_scatter(res_v, [b], acc)
        return carry

    lax.fori_loop(0, BPW // LANES, chunk_body, 0)

    pltpu.sync_copy(res_v, out_hbm.at[pl.ds(base, BPW)])


def kernel(user, item, user_emb_table, item_emb_table):
    return _mf_sc(user.astype(jnp.int32), item.astype(jnp.int32),
                  user_emb_table, item_emb_table)


# speedup vs baseline: 1.0007x; 1.0007x over previous
import functools

import jax
import jax.numpy as jnp
from jax import lax
from jax.experimental import pallas as pl
from jax.experimental.pallas import tpu as pltpu
from jax.experimental.pallas import tpu_sc as plsc

BATCH = 16384
DIM = 32
NC = 2
NS = 16
LANES = 16
NW = NC * NS
BPW = BATCH // NW
CH = 128
NCHUNK = BPW // CH
ROWS_PER_LINE = 4
LINE = DIM * ROWS_PER_LINE

_mesh = plsc.VectorSubcoreMesh(core_axis_name="c", subcore_axis_name="s")


@functools.partial(
    pl.kernel,
    out_type=jax.ShapeDtypeStruct((BATCH,), jnp.float32),
    mesh=_mesh,
    compiler_params=pltpu.CompilerParams(needs_layout_passes=False),
    scratch_types=[
        pltpu.VMEM((NCHUNK, CH), jnp.int32),
        pltpu.VMEM((NCHUNK, CH), jnp.int32),
        pltpu.VMEM((NCHUNK, CH), jnp.int32),
        pltpu.VMEM((NCHUNK, CH), jnp.int32),
        pltpu.VMEM((2, CH, LINE), jnp.float32),
        pltpu.VMEM((2, CH, LINE), jnp.float32),
        pltpu.VMEM((BPW,), jnp.float32),
        pltpu.SemaphoreType.DMA,
        pltpu.SemaphoreType.DMA,
        pltpu.SemaphoreType.DMA,
        pltpu.SemaphoreType.DMA,
        pltpu.SemaphoreType.DMA,
    ],
)
def _mf_sc(user_hbm, item_hbm, utab_hbm, itab_hbm, out_hbm,
           uidx_v, iidx_v, ugidx_v, igidx_v, ubuf, ibuf, res_v,
           sem_idx, sem_u0, sem_u1, sem_i0, sem_i1):
    wid = lax.axis_index("s") * NC + lax.axis_index("c")
    base = wid * BPW

    idx_copies = []
    for j in range(NCHUNK):
        off = base + j * CH
        idx_copies.append(pltpu.async_copy(
            user_hbm.at[pl.ds(off, CH)], uidx_v.at[j], sem_idx))
        idx_copies.append(pltpu.async_copy(
            item_hbm.at[pl.ds(off, CH)], iidx_v.at[j], sem_idx))
    for c in idx_copies:
        c.wait()

    lane = lax.iota(jnp.int32, LANES)
    for j in range(NCHUNK):
        jf = jnp.full((LANES,), j, jnp.int32)
        for q in range(CH // LANES):
            kq = lane + q * LANES
            ru = plsc.load_gather(uidx_v, [jf, kq])
            ri = plsc.load_gather(iidx_v, [jf, kq])
            plsc.store_scatter(ugidx_v, [jf, kq],
                               lax.shift_right_logical(ru, 2))
            plsc.store_scatter(igidx_v, [jf, kq],
                               lax.shift_right_logical(ri, 2))

    sem_u = (sem_u0, sem_u1)
    sem_i = (sem_i0, sem_i1)

    def fire(j):
        s = j & 1
        return (pltpu.async_copy(utab_hbm.at[ugidx_v.at[j]], ubuf.at[s],
                                 sem_u[s]),
                pltpu.async_copy(itab_hbm.at[igidx_v.at[j]], ibuf.at[s],
                                 sem_i[s]))

    pending = fire(0)
    for j in range(NCHUNK):
        cu_, ci_ = pending
        if j + 1 < NCHUNK:
            nxt = fire(j + 1)
        cu_.wait()
        ci_.wait()
        if j + 1 < NCHUNK:
            pending = nxt

        s = j & 1
        ub2 = ubuf.at[s]
        ib2 = ibuf.at[s]
        jf = jnp.full((LANES,), j, jnp.int32)

        def qbody(q, carry, ub2=ub2, ib2=ib2, jf=jf):
            kq = lane + q * LANES
            ru = plsc.load_gather(uidx_v, [jf, kq])
            ri = plsc.load_gather(iidx_v, [jf, kq])
            ucol = lax.shift_left(ru & 3, 5)
            icol = lax.shift_left(ri & 3, 5)
            acc = jnp.zeros((LANES,), jnp.float32)
            for d in range(DIM):
                u = plsc.load_gather(ub2, [kq, ucol + d])
                v = plsc.load_gather(ib2, [kq, icol + d])
                acc = acc + u * v
            plsc.store_scatter(res_v, [jf * CH + kq], acc)
            return carry

        lax.fori_loop(0, CH // LANES, qbody, 0)

    pltpu.sync_copy(res_v, out_hbm.at[pl.ds(base, BPW)])


def kernel(user, item, user_emb_table, item_emb_table):
    utab = user_emb_table.reshape(user_emb_table.shape[0] // ROWS_PER_LINE,
                                  LINE)
    itab = item_emb_table.reshape(item_emb_table.shape[0] // ROWS_PER_LINE,
                                  LINE)
    return _mf_sc(user.astype(jnp.int32), item.astype(jnp.int32), utab, itab)
